# baseline (speedup 1.0000x reference)
import functools
import math

import jax
import jax.numpy as jnp
from jax import lax
from jax.experimental import pallas as pl
from jax.experimental.pallas import tpu as pltpu
from jax.experimental.pallas import tpu_sc as plsc

N = 50000
E = 800000
H = 128
T = 300
B = 64

NTILES = 16
NCORES = 2
DUMP = 16
EPT = E // NTILES
WSCAN = 2000
NWIN = EPT // WSCAN
VPW = WSCAN // 16
FB = 64
NBLK = (WSCAN + FB - 1) // FB


def _sc_agg_body(from_emb, npc, rchunk, srcs, dsts, h, ones_hbm, zagg, zdeg,
                 x_hbm,
                 agg_out, deg_out,
                 sp_agg, sp_deg, win_a, win_b, lsrc, loff, rows_v, ones_v,
                 degb, dsem, rows_w, dsem2, ssem, gsem, emb_sp=None,
                 tok_a=None, tok_b=None, dsem3=None):
    c = lax.axis_index("c")
    s = lax.axis_index("s")
    e0 = s * EPT
    pltpu.sync_copy(ones_hbm, ones_v)
    if from_emb:
        @pl.when(s == 0)
        def _():
            pltpu.sync_copy(h, emb_sp)
        plsc.subcore_barrier()
    iota = lax.iota(jnp.int32, 16)

    def process(sw, dw, base):
        def scan_vreg(i, cnt):
            sv = sw[pl.ds(i * 16, 16)]
            dv = dw[pl.ds(i * 16, 16)]
            m = (dv >= base) & (dv < base + rchunk)
            incl = plsc.cumsum(m.astype(jnp.int32))
            pos = cnt + incl - 1
            hi = lax.shift_right_logical(pos, 6)
            lo = pos & 63
            plsc.store_scatter(loff, [hi, lo], dv - base, mask=m)
            plsc.store_scatter(lsrc, [hi, lo], sv, mask=m)
            return cnt + jnp.max(incl)

        cnt = lax.fori_loop(0, VPW, scan_vreg, jnp.int32(0), unroll=2)
        kpad = (cnt + 63) & jnp.int32(-64)
        for j in range(4):
            p = cnt + j * 16 + iota
            pm = p < kpad
            plsc.store_scatter(loff, [lax.shift_right_logical(p, 6), p & 63],
                               rchunk + (p & 15), mask=pm)
            plsc.store_scatter(lsrc, [lax.shift_right_logical(p, 6), p & 63],
                               p & 63, mask=pm)

        nblk = lax.shift_right_logical(kpad, 6)

        def wait_scatter(rows_ref, j):
            pltpu.make_async_copy(rows_ref, sp_agg.at[loff.at[j]],
                                  ssem).wait()

        if from_emb:
            @pl.when(nblk > 0)
            def _():
                pltpu.async_copy(x_hbm.at[lsrc.at[0]], tok_a, dsem)

            def flush(j, carry):
                def step(tok_cur, sem_cur, tok_nxt, sem_nxt, rows_cur,
                         rows_prv):
                    @pl.when(j + 1 < nblk)
                    def _():
                        pltpu.async_copy(x_hbm.at[lsrc.at[j + 1]], tok_nxt,
                                         sem_nxt)
                    pltpu.make_async_copy(x_hbm.at[pl.ds(0, FB)], tok_cur,
                                          sem_cur).wait()
                    @pl.when(j >= 1)
                    def _():
                        wait_scatter(rows_prv, j - 1)
                    pltpu.async_copy(emb_sp.at[tok_cur], rows_cur,
                                     dsem3).wait()
                    pltpu.async_copy(rows_cur, sp_agg.at[loff.at[j]], ssem,
                                     add=True)
                    pltpu.async_copy(ones_v, sp_deg.at[loff.at[j]], gsem,
                                     add=True)

                @pl.when((j & 1) == 0)
                def _():
                    step(tok_a, dsem, tok_b, dsem2, rows_v, rows_w)

                @pl.when((j & 1) == 1)
                def _():
                    step(tok_b, dsem2, tok_a, dsem, rows_w, rows_v)
                return carry
        else:
            @pl.when(nblk > 0)
            def _():
                pltpu.async_copy(h.at[lsrc.at[0]], rows_v, dsem)

            def flush(j, carry):
                def step(rows_cur, sem_cur, rows_nxt, sem_nxt):
                    @pl.when(j + 1 < nblk)
                    def _():
                        pltpu.async_copy(h.at[lsrc.at[j + 1]], rows_nxt,
                                         sem_nxt)
                    pltpu.make_async_copy(h.at[pl.ds(0, FB)], rows_cur,
                                          sem_cur).wait()
                    pltpu.sync_copy(rows_cur, sp_agg.at[loff.at[j]],
                                    add=True)
                    pltpu.sync_copy(ones_v, sp_deg.at[loff.at[j]], add=True)

                @pl.when((j & 1) == 0)
                def _():
                    step(rows_v, dsem, rows_w, dsem2)

                @pl.when((j & 1) == 1)
                def _():
                    step(rows_w, dsem2, rows_v, dsem)
                return carry

        lax.fori_loop(0, nblk, flush, 0)

        if from_emb:
            @pl.when(nblk > 0)
            def _():
                @pl.when((nblk & 1) == 1)
                def _():
                    wait_scatter(rows_v, nblk - 1)

                @pl.when((nblk & 1) == 0)
                def _():
                    wait_scatter(rows_w, nblk - 1)

            def drain_deg(j, carry):
                pltpu.make_async_copy(ones_v, sp_deg.at[loff.at[0]],
                                      gsem).wait()
                return carry

            lax.fori_loop(0, nblk, drain_deg, 0)

    share = rchunk // NTILES
    for k in range(npc):
        base = (npc * c + k) * rchunk
        pltpu.sync_copy(zagg, sp_agg.at[pl.ds(s * share, share)])
        pltpu.sync_copy(zdeg, degb)
        pltpu.sync_copy(degb, sp_deg.at[pl.ds(s * share, share)])
        plsc.subcore_barrier()

        def window(w, carry):
            off = e0 + w * WSCAN
            pltpu.sync_copy(srcs.at[pl.ds(off, WSCAN)], win_a)
            pltpu.sync_copy(dsts.at[pl.ds(off, WSCAN)], win_b)
            process(win_a, win_b, base)
            process(win_b, win_a, base)
            return carry

        lax.fori_loop(0, NWIN, window, 0)
        plsc.subcore_barrier()
        pltpu.sync_copy(sp_agg.at[pl.ds(s * share, share)],
                        agg_out.at[pl.ds(base + s * share, share)])
        pltpu.sync_copy(sp_deg.at[pl.ds(s * share, share)], degb)
        pltpu.sync_copy(degb, deg_out.at[pl.ds(base + s * share, share)])
        plsc.subcore_barrier()


def _sc_agg(srcs, dsts, h, x=None):
    from_emb = x is not None
    npc = 3 if from_emb else 2
    rchunk = 8448 if from_emb else 12544
    n2 = NCORES * npc * rchunk
    share = rchunk // NTILES
    ones_arr = jnp.ones((FB,), jnp.float32)
    zagg = jnp.zeros((share, H), jnp.float32)
    zdeg = jnp.zeros((share,), jnp.float32)
    if x is None:
        x = jnp.zeros((8,), jnp.int32)
    scratch = [
        pltpu.VMEM_SHARED((rchunk + DUMP, H), jnp.float32),
        pltpu.VMEM_SHARED((rchunk + DUMP,), jnp.float32),
        pltpu.VMEM((WSCAN,), jnp.int32),
        pltpu.VMEM((WSCAN,), jnp.int32),
        pltpu.VMEM((NBLK, FB), jnp.int32),
        pltpu.VMEM((NBLK, FB), jnp.int32),
        pltpu.VMEM((FB, H), jnp.float32),
        pltpu.VMEM((FB,), jnp.float32),
        pltpu.VMEM((share,), jnp.float32),
        pltpu.SemaphoreType.DMA,
    ]
    scratch += [pltpu.VMEM((FB, H), jnp.float32),
                pltpu.SemaphoreType.DMA,
                pltpu.SemaphoreType.DMA,
                pltpu.SemaphoreType.DMA]
    if from_emb:
        scratch += [pltpu.VMEM_SHARED((T2, H), jnp.float32),
                    pltpu.VMEM((FB,), jnp.int32),
                    pltpu.VMEM((FB,), jnp.int32),
                    pltpu.SemaphoreType.DMA]
    mesh = plsc.VectorSubcoreMesh(core_axis_name="c", subcore_axis_name="s")
    f = pl.kernel(
        functools.partial(_sc_agg_body, from_emb, npc, rchunk),
        out_type=(jax.ShapeDtypeStruct((n2, H), jnp.float32),
                  jax.ShapeDtypeStruct((n2,), jnp.float32)),
        mesh=mesh,
        compiler_params=pltpu.CompilerParams(needs_layout_passes=False),
        scratch_types=scratch,
    )
    return f(srcs, dsts, h, ones_arr, zagg, zdeg, x)


BN = 400
NB = N // BN
T2 = 304


def _embed_body(x_ref, emb_ref, o_ref):
    xb = x_ref[0]
    it = lax.broadcasted_iota(jnp.int32, (T2, BN), 0)
    oh = (it == xb).astype(jnp.float32)
    o_ref[...] = lax.dot_general(oh, emb_ref[...], (((0,), (0,)), ((), ())),
                                 preferred_element_type=jnp.float32)


def _embed(x, emb):
    x3 = x.reshape(NB, 1, BN)
    emb_p = jnp.pad(emb, ((0, T2 - T), (0, 0)))
    return pl.pallas_call(
        _embed_body,
        grid=(NB,),
        in_specs=[pl.BlockSpec((1, 1, BN), lambda i: (i, 0, 0)),
                  pl.BlockSpec((T2, H), lambda i: (0, 0))],
        out_specs=pl.BlockSpec((BN, H), lambda i: (i, 0)),
        out_shape=jax.ShapeDtypeStruct((N, H), jnp.float32),
    )(x3, emb_p)


def _layer_body(agg_ref, deg_ref, h_ref, w_ref, b_ref, o_ref):
    a = agg_ref[...] * lax.rsqrt(jnp.clip(deg_ref[...], 1.0, None))
    z = lax.dot_general(a, w_ref[...], (((1,), (1,)), ((), ())),
                        preferred_element_type=jnp.float32)
    z = z + b_ref[...] + h_ref[...]
    o_ref[...] = 0.5 * z * (1.0 + lax.erf(z * (1.0 / math.sqrt(2.0))))


def _layer(agg_p, deg_p, h, w, b):
    return pl.pallas_call(
        _layer_body,
        grid=(NB,),
        in_specs=[pl.BlockSpec((BN, H), lambda i: (i, 0)),
                  pl.BlockSpec((BN, 1), lambda i: (i, 0)),
                  pl.BlockSpec((BN, H), lambda i: (i, 0)),
                  pl.BlockSpec((H, H), lambda i: (0, 0)),
                  pl.BlockSpec((1, H), lambda i: (0, 0))],
        out_specs=pl.BlockSpec((BN, H), lambda i: (i, 0)),
        out_shape=jax.ShapeDtypeStruct((N, H), jnp.float32),
    )(agg_p, deg_p.reshape(-1, 1), h, w, b.reshape(1, H))


def _pool_body(batch_ref, h_ref, wo_ref, bo_ref, g_ref, be_ref, o_ref,
               acc, cnt):
    i = pl.program_id(0)

    @pl.when(i == 0)
    def _():
        acc[...] = jnp.zeros_like(acc)
        cnt[...] = jnp.zeros_like(cnt)

    bb = batch_ref[0]
    seg = lax.broadcasted_iota(jnp.int32, (B, BN), 0)
    m = (seg == bb).astype(jnp.float32)
    acc[...] += lax.dot_general(m, h_ref[...], (((1,), (0,)), ((), ())),
                                preferred_element_type=jnp.float32)
    cnt[...] += jnp.sum(m, axis=1, keepdims=True)

    @pl.when(i == NB - 1)
    def _():
        gf = acc[...] / jnp.clip(cnt[...], 1.0, None)
        o = lax.dot_general(gf, wo_ref[...], (((1,), (1,)), ((), ())),
                            preferred_element_type=jnp.float32) + bo_ref[...]
        mu = jnp.mean(o, axis=1, keepdims=True)
        var = jnp.mean((o - mu) ** 2, axis=1, keepdims=True)
        o_ref[...] = (o - mu) * lax.rsqrt(var + 1e-5) * g_ref[...] + be_ref[...]


def _pool(h2, batch, wo, bo, gamma, beta):
    batch3 = batch.reshape(NB, 1, BN)
    return pl.pallas_call(
        _pool_body,
        grid=(NB,),
        in_specs=[pl.BlockSpec((1, 1, BN), lambda i: (i, 0, 0)),
                  pl.BlockSpec((BN, H), lambda i: (i, 0)),
                  pl.BlockSpec((H, H), lambda i: (0, 0)),
                  pl.BlockSpec((1, H), lambda i: (0, 0)),
                  pl.BlockSpec((1, H), lambda i: (0, 0)),
                  pl.BlockSpec((1, H), lambda i: (0, 0))],
        out_specs=pl.BlockSpec((B, H), lambda i: (0, 0)),
        out_shape=jax.ShapeDtypeStruct((B, H), jnp.float32),
        scratch_shapes=[pltpu.VMEM((B, H), jnp.float32),
                        pltpu.VMEM((B, 1), jnp.float32)],
    )(batch3, h2, wo, bo.reshape(1, H), gamma.reshape(1, H),
      beta.reshape(1, H))


def kernel(x, edge_index, batch, batch_size, emb, W1, b1, W2, b2, Wo, bo,
           gamma, beta):
    srcs = edge_index[0]
    dsts = edge_index[1]
    h0 = _embed(x, emb)
    emb_p = jnp.pad(emb, ((0, T2 - T), (0, 0)))
    agg1, deg = _sc_agg(srcs, dsts, emb_p, x=x)
    h1 = _layer(agg1, deg, h0, W1, b1)
    agg2, _ = _sc_agg(srcs, dsts, h1)
    h2 = _layer(agg2, deg, h1, W2, b2)
    return _pool(h2, batch, Wo, bo, gamma, beta)


# speedup vs baseline: 1.2062x; 1.2062x over previous
import functools
import math

import jax
import jax.numpy as jnp
from jax import lax
from jax.experimental import pallas as pl
from jax.experimental.pallas import tpu as pltpu
from jax.experimental.pallas import tpu_sc as plsc

N = 50000
E = 800000
H = 128
T = 300
B = 64

NTILES = 16
NCORES = 2
DUMP = 16
EPT = E // NTILES
WSCAN = 2000
NWIN = EPT // WSCAN
VPW = WSCAN // 16
FB = 64
NBLK = (WSCAN + FB - 1) // FB


def _sc_agg_body(from_emb, npc, rchunk, srcs, dsts, h, ones_hbm, zagg, zdeg,
                 x_hbm,
                 agg_out, deg_out,
                 sp_agg, sp_deg, win_a, win_b, lsrc, loff, rows_v, ones_v,
                 degb, dsem, rows_w, dsem2, ssem, gsem, emb_sp=None,
                 tok_a=None, tok_b=None, dsem3=None):
    c = lax.axis_index("c")
    s = lax.axis_index("s")
    e0 = s * EPT
    pltpu.sync_copy(ones_hbm, ones_v)
    if from_emb:
        @pl.when(s == 0)
        def _():
            pltpu.sync_copy(h, emb_sp)
        plsc.subcore_barrier()
    iota = lax.iota(jnp.int32, 16)

    def process(sw, dw, base):
        def scan_vreg(i, cnt):
            sv = sw[pl.ds(i * 16, 16)]
            dv = dw[pl.ds(i * 16, 16)]
            m = (dv >= base) & (dv < base + rchunk)
            incl = plsc.cumsum(m.astype(jnp.int32))
            pos = cnt + incl - 1
            hi = lax.shift_right_logical(pos, 6)
            lo = pos & 63
            plsc.store_scatter(loff, [hi, lo], dv - base, mask=m)
            plsc.store_scatter(lsrc, [hi, lo], sv, mask=m)
            return cnt + jnp.max(incl)

        cnt = lax.fori_loop(0, VPW, scan_vreg, jnp.int32(0), unroll=2)
        kpad = (cnt + 63) & jnp.int32(-64)
        for j in range(4):
            p = cnt + j * 16 + iota
            pm = p < kpad
            plsc.store_scatter(loff, [lax.shift_right_logical(p, 6), p & 63],
                               rchunk + (p & 15), mask=pm)
            plsc.store_scatter(lsrc, [lax.shift_right_logical(p, 6), p & 63],
                               p & 63, mask=pm)

        nblk = lax.shift_right_logical(kpad, 6)

        def wait_scatter(rows_ref, j):
            pltpu.make_async_copy(rows_ref, sp_agg.at[loff.at[j]],
                                  ssem).wait()

        if from_emb:
            @pl.when(nblk > 0)
            def _():
                pltpu.async_copy(x_hbm.at[lsrc.at[0]], tok_a, dsem)

            def flush(j, carry):
                def step(tok_cur, sem_cur, tok_nxt, sem_nxt, rows_cur,
                         rows_prv):
                    @pl.when(j + 1 < nblk)
                    def _():
                        pltpu.async_copy(x_hbm.at[lsrc.at[j + 1]], tok_nxt,
                                         sem_nxt)
                    pltpu.make_async_copy(x_hbm.at[pl.ds(0, FB)], tok_cur,
                                          sem_cur).wait()
                    @pl.when(j >= 1)
                    def _():
                        wait_scatter(rows_prv, j - 1)
                    pltpu.async_copy(emb_sp.at[tok_cur], rows_cur,
                                     dsem3).wait()
                    pltpu.async_copy(rows_cur, sp_agg.at[loff.at[j]], ssem,
                                     add=True)
                    pltpu.async_copy(ones_v, sp_deg.at[loff.at[j]], gsem,
                                     add=True)

                @pl.when((j & 1) == 0)
                def _():
                    step(tok_a, dsem, tok_b, dsem2, rows_v, rows_w)

                @pl.when((j & 1) == 1)
                def _():
                    step(tok_b, dsem2, tok_a, dsem, rows_w, rows_v)
                return carry
        else:
            @pl.when(nblk > 0)
            def _():
                pltpu.async_copy(h.at[lsrc.at[0]], rows_v, dsem)

            def flush(j, carry):
                def step(rows_cur, sem_cur, rows_nxt, sem_nxt):
                    @pl.when(j + 1 < nblk)
                    def _():
                        pltpu.async_copy(h.at[lsrc.at[j + 1]], rows_nxt,
                                         sem_nxt)
                    pltpu.make_async_copy(h.at[pl.ds(0, FB)], rows_cur,
                                          sem_cur).wait()
                    pltpu.sync_copy(rows_cur, sp_agg.at[loff.at[j]],
                                    add=True)
                    pltpu.sync_copy(ones_v, sp_deg.at[loff.at[j]], add=True)

                @pl.when((j & 1) == 0)
                def _():
                    step(rows_v, dsem, rows_w, dsem2)

                @pl.when((j & 1) == 1)
                def _():
                    step(rows_w, dsem2, rows_v, dsem)
                return carry

        lax.fori_loop(0, nblk, flush, 0)

        if from_emb:
            @pl.when(nblk > 0)
            def _():
                @pl.when((nblk & 1) == 1)
                def _():
                    wait_scatter(rows_v, nblk - 1)

                @pl.when((nblk & 1) == 0)
                def _():
                    wait_scatter(rows_w, nblk - 1)

            def drain_deg(j, carry):
                pltpu.make_async_copy(ones_v, sp_deg.at[loff.at[0]],
                                      gsem).wait()
                return carry

            lax.fori_loop(0, nblk, drain_deg, 0)

    share = rchunk // NTILES
    for k in range(npc):
        base = (npc * c + k) * rchunk
        pltpu.sync_copy(zagg, sp_agg.at[pl.ds(s * share, share)])
        pltpu.sync_copy(zdeg, degb)
        pltpu.sync_copy(degb, sp_deg.at[pl.ds(s * share, share)])
        plsc.subcore_barrier()

        def window(w, carry):
            off = e0 + w * WSCAN
            pltpu.sync_copy(srcs.at[pl.ds(off, WSCAN)], win_a)
            pltpu.sync_copy(dsts.at[pl.ds(off, WSCAN)], win_b)
            process(win_a, win_b, base)
            process(win_b, win_a, base)
            return carry

        lax.fori_loop(0, NWIN, window, 0)
        plsc.subcore_barrier()
        pltpu.sync_copy(sp_agg.at[pl.ds(s * share, share)],
                        agg_out.at[pl.ds(base + s * share, share)])
        pltpu.sync_copy(sp_deg.at[pl.ds(s * share, share)], degb)
        pltpu.sync_copy(degb, deg_out.at[pl.ds(base + s * share, share)])
        plsc.subcore_barrier()


def _sc_agg(srcs, dsts, h, x=None):
    from_emb = x is not None
    npc = 3 if from_emb else 2
    rchunk = 8448 if from_emb else 12544
    n2 = NCORES * npc * rchunk
    share = rchunk // NTILES
    ones_arr = jnp.ones((FB,), jnp.float32)
    zagg = jnp.zeros((share, H), jnp.float32)
    zdeg = jnp.zeros((share,), jnp.float32)
    if x is None:
        x = jnp.zeros((8,), jnp.int32)
    scratch = [
        pltpu.VMEM_SHARED((rchunk + DUMP, H), jnp.float32),
        pltpu.VMEM_SHARED((rchunk + DUMP,), jnp.float32),
        pltpu.VMEM((WSCAN,), jnp.int32),
        pltpu.VMEM((WSCAN,), jnp.int32),
        pltpu.VMEM((NBLK, FB), jnp.int32),
        pltpu.VMEM((NBLK, FB), jnp.int32),
        pltpu.VMEM((FB, H), jnp.float32),
        pltpu.VMEM((FB,), jnp.float32),
        pltpu.VMEM((share,), jnp.float32),
        pltpu.SemaphoreType.DMA,
    ]
    scratch += [pltpu.VMEM((FB, H), jnp.float32),
                pltpu.SemaphoreType.DMA,
                pltpu.SemaphoreType.DMA,
                pltpu.SemaphoreType.DMA]
    if from_emb:
        scratch += [pltpu.VMEM_SHARED((T2, H), jnp.float32),
                    pltpu.VMEM((FB,), jnp.int32),
                    pltpu.VMEM((FB,), jnp.int32),
                    pltpu.SemaphoreType.DMA]
    mesh = plsc.VectorSubcoreMesh(core_axis_name="c", subcore_axis_name="s")
    f = pl.kernel(
        functools.partial(_sc_agg_body, from_emb, npc, rchunk),
        out_type=(jax.ShapeDtypeStruct((n2, H), jnp.float32),
                  jax.ShapeDtypeStruct((n2,), jnp.float32)),
        mesh=mesh,
        compiler_params=pltpu.CompilerParams(needs_layout_passes=False),
        scratch_types=scratch,
    )
    return f(srcs, dsts, h, ones_arr, zagg, zdeg, x)


BN = 400
NB = N // BN
T2 = 304


def _embed_body(x_ref, emb_ref, o_ref):
    xb = x_ref[0]
    it = lax.broadcasted_iota(jnp.int32, (T2, BN), 0)
    oh = (it == xb).astype(jnp.float32)
    o_ref[...] = lax.dot_general(oh, emb_ref[...], (((0,), (0,)), ((), ())),
                                 preferred_element_type=jnp.float32)


def _embed(x, emb):
    x3 = x.reshape(NB, 1, BN)
    emb_p = jnp.pad(emb, ((0, T2 - T), (0, 0)))
    return pl.pallas_call(
        _embed_body,
        grid=(NB,),
        in_specs=[pl.BlockSpec((1, 1, BN), lambda i: (i, 0, 0)),
                  pl.BlockSpec((T2, H), lambda i: (0, 0))],
        out_specs=pl.BlockSpec((BN, H), lambda i: (i, 0)),
        out_shape=jax.ShapeDtypeStruct((N, H), jnp.float32),
    )(x3, emb_p)


def _layer_body(agg_ref, deg_ref, h_ref, w_ref, b_ref, o_ref):
    a = agg_ref[...] * lax.rsqrt(jnp.clip(deg_ref[...], 1.0, None))
    z = lax.dot_general(a, w_ref[...], (((1,), (1,)), ((), ())),
                        preferred_element_type=jnp.float32)
    z = z + b_ref[...] + h_ref[...]
    o_ref[...] = 0.5 * z * (1.0 + lax.erf(z * (1.0 / math.sqrt(2.0))))


def _layer(agg_p, deg_p, h, w, b):
    return pl.pallas_call(
        _layer_body,
        grid=(NB,),
        in_specs=[pl.BlockSpec((BN, H), lambda i: (i, 0)),
                  pl.BlockSpec((BN, 1), lambda i: (i, 0)),
                  pl.BlockSpec((BN, H), lambda i: (i, 0)),
                  pl.BlockSpec((H, H), lambda i: (0, 0)),
                  pl.BlockSpec((1, H), lambda i: (0, 0))],
        out_specs=pl.BlockSpec((BN, H), lambda i: (i, 0)),
        out_shape=jax.ShapeDtypeStruct((N, H), jnp.float32),
    )(agg_p, deg_p.reshape(-1, 1), h, w, b.reshape(1, H))


def _pool_body(batch_ref, h_ref, wo_ref, bo_ref, g_ref, be_ref, o_ref,
               acc, cnt):
    i = pl.program_id(0)

    @pl.when(i == 0)
    def _():
        acc[...] = jnp.zeros_like(acc)
        cnt[...] = jnp.zeros_like(cnt)

    bb = batch_ref[0]
    seg = lax.broadcasted_iota(jnp.int32, (B, BN), 0)
    m = (seg == bb).astype(jnp.float32)
    acc[...] += lax.dot_general(m, h_ref[...], (((1,), (0,)), ((), ())),
                                preferred_element_type=jnp.float32)
    cnt[...] += jnp.sum(m, axis=1, keepdims=True)

    @pl.when(i == NB - 1)
    def _():
        gf = acc[...] / jnp.clip(cnt[...], 1.0, None)
        o = lax.dot_general(gf, wo_ref[...], (((1,), (1,)), ((), ())),
                            preferred_element_type=jnp.float32) + bo_ref[...]
        mu = jnp.mean(o, axis=1, keepdims=True)
        var = jnp.mean((o - mu) ** 2, axis=1, keepdims=True)
        o_ref[...] = (o - mu) * lax.rsqrt(var + 1e-5) * g_ref[...] + be_ref[...]


def _pool(h2, batch, wo, bo, gamma, beta):
    batch3 = batch.reshape(NB, 1, BN)
    return pl.pallas_call(
        _pool_body,
        grid=(NB,),
        in_specs=[pl.BlockSpec((1, 1, BN), lambda i: (i, 0, 0)),
                  pl.BlockSpec((BN, H), lambda i: (i, 0)),
                  pl.BlockSpec((H, H), lambda i: (0, 0)),
                  pl.BlockSpec((1, H), lambda i: (0, 0)),
                  pl.BlockSpec((1, H), lambda i: (0, 0)),
                  pl.BlockSpec((1, H), lambda i: (0, 0))],
        out_specs=pl.BlockSpec((B, H), lambda i: (0, 0)),
        out_shape=jax.ShapeDtypeStruct((B, H), jnp.float32),
        scratch_shapes=[pltpu.VMEM((B, H), jnp.float32),
                        pltpu.VMEM((B, 1), jnp.float32)],
    )(batch3, h2, wo, bo.reshape(1, H), gamma.reshape(1, H),
      beta.reshape(1, H))


def kernel(x, edge_index, batch, batch_size, emb, W1, b1, W2, b2, Wo, bo,
           gamma, beta):
    srcs = edge_index[0]
    dsts = edge_index[1]
    h0 = _embed(x, emb)
    agg1, deg = _sc_agg(srcs, dsts, h0)
    h1 = _layer(agg1, deg, h0, W1, b1)
    agg2, _ = _sc_agg(srcs, dsts, h1)
    h2 = _layer(agg2, deg, h1, W2, b2)
    return _pool(h2, batch, Wo, bo, gamma, beta)


# speedup vs baseline: 1.2428x; 1.0303x over previous
import functools
import math

import jax
import jax.numpy as jnp
from jax import lax
from jax.experimental import pallas as pl
from jax.experimental.pallas import tpu as pltpu
from jax.experimental.pallas import tpu_sc as plsc

N = 50000
E = 800000
H = 128
T = 300
B = 64

NTILES = 16
NCORES = 2
DUMP = 16
EPT = E // NTILES
WSCAN = 2000
NWIN = EPT // WSCAN
VPW = WSCAN // 16
FB = 64
NBLK = (WSCAN + FB - 1) // FB


def _sc_agg_body(from_emb, npc, rchunk, srcs, dsts, h, ones_hbm, zagg, zdeg,
                 x_hbm,
                 agg_out, deg_out,
                 sp_agg, sp_deg, win_a, win_b, lsrc, loff, rows_v, ones_v,
                 degb, dsem, rows_w, dsem2, ssem, gsem, emb_sp=None,
                 tok_a=None, tok_b=None, dsem3=None):
    c = lax.axis_index("c")
    s = lax.axis_index("s")
    e0 = s * EPT
    pltpu.sync_copy(ones_hbm, ones_v)
    if from_emb:
        @pl.when(s == 0)
        def _():
            pltpu.sync_copy(h, emb_sp)
        plsc.subcore_barrier()
    iota = lax.iota(jnp.int32, 16)

    def process(sw, dw, base):
        def scan_vreg(i, cnt):
            sv = sw[pl.ds(i * 16, 16)]
            dv = dw[pl.ds(i * 16, 16)]
            m = (dv >= base) & (dv < base + rchunk)
            incl = plsc.cumsum(m.astype(jnp.int32))
            pos = cnt + incl - 1
            hi = lax.shift_right_logical(pos, 6)
            lo = pos & 63
            plsc.store_scatter(loff, [hi, lo], dv - base, mask=m)
            plsc.store_scatter(lsrc, [hi, lo], sv, mask=m)
            return cnt + jnp.max(incl)

        cnt = lax.fori_loop(0, VPW, scan_vreg, jnp.int32(0), unroll=2)
        kpad = (cnt + 63) & jnp.int32(-64)
        for j in range(4):
            p = cnt + j * 16 + iota
            pm = p < kpad
            plsc.store_scatter(loff, [lax.shift_right_logical(p, 6), p & 63],
                               rchunk + (p & 15), mask=pm)
            plsc.store_scatter(lsrc, [lax.shift_right_logical(p, 6), p & 63],
                               p & 63, mask=pm)

        nblk = lax.shift_right_logical(kpad, 6)

        def wait_scatter(rows_ref, j):
            pltpu.make_async_copy(rows_ref, sp_agg.at[loff.at[j]],
                                  ssem).wait()

        if from_emb:
            @pl.when(nblk > 0)
            def _():
                pltpu.async_copy(x_hbm.at[lsrc.at[0]], tok_a, dsem)

            def flush(j, carry):
                def step(tok_cur, sem_cur, tok_nxt, sem_nxt, rows_cur,
                         rows_prv):
                    @pl.when(j + 1 < nblk)
                    def _():
                        pltpu.async_copy(x_hbm.at[lsrc.at[j + 1]], tok_nxt,
                                         sem_nxt)
                    pltpu.make_async_copy(x_hbm.at[pl.ds(0, FB)], tok_cur,
                                          sem_cur).wait()
                    @pl.when(j >= 1)
                    def _():
                        wait_scatter(rows_prv, j - 1)
                    pltpu.async_copy(emb_sp.at[tok_cur], rows_cur,
                                     dsem3).wait()
                    pltpu.async_copy(rows_cur, sp_agg.at[loff.at[j]], ssem,
                                     add=True)
                    pltpu.async_copy(ones_v, sp_deg.at[loff.at[j]], gsem,
                                     add=True)

                @pl.when((j & 1) == 0)
                def _():
                    step(tok_a, dsem, tok_b, dsem2, rows_v, rows_w)

                @pl.when((j & 1) == 1)
                def _():
                    step(tok_b, dsem2, tok_a, dsem, rows_w, rows_v)
                return carry
        else:
            @pl.when(nblk > 0)
            def _():
                pltpu.async_copy(h.at[lsrc.at[0]], rows_v, dsem)

            def flush(j, carry):
                def step(rows_cur, sem_cur, rows_nxt, sem_nxt):
                    @pl.when(j + 1 < nblk)
                    def _():
                        pltpu.async_copy(h.at[lsrc.at[j + 1]], rows_nxt,
                                         sem_nxt)
                    pltpu.make_async_copy(h.at[pl.ds(0, FB)], rows_cur,
                                          sem_cur).wait()
                    pltpu.sync_copy(rows_cur, sp_agg.at[loff.at[j]],
                                    add=True)
                    pltpu.sync_copy(ones_v, sp_deg.at[loff.at[j]], add=True)

                @pl.when((j & 1) == 0)
                def _():
                    step(rows_v, dsem, rows_w, dsem2)

                @pl.when((j & 1) == 1)
                def _():
                    step(rows_w, dsem2, rows_v, dsem)
                return carry

        lax.fori_loop(0, nblk, flush, 0)

        if from_emb:
            @pl.when(nblk > 0)
            def _():
                @pl.when((nblk & 1) == 1)
                def _():
                    wait_scatter(rows_v, nblk - 1)

                @pl.when((nblk & 1) == 0)
                def _():
                    wait_scatter(rows_w, nblk - 1)

            def drain_deg(j, carry):
                pltpu.make_async_copy(ones_v, sp_deg.at[loff.at[0]],
                                      gsem).wait()
                return carry

            lax.fori_loop(0, nblk, drain_deg, 0)

    share = rchunk // NTILES
    for k in range(npc):
        base = (npc * c + k) * rchunk
        pltpu.sync_copy(zagg, sp_agg.at[pl.ds(s * share, share)])
        pltpu.sync_copy(zdeg, degb)
        pltpu.sync_copy(degb, sp_deg.at[pl.ds(s * share, share)])
        plsc.subcore_barrier()

        def window(w, carry):
            off = e0 + w * WSCAN
            pltpu.sync_copy(srcs.at[pl.ds(off, WSCAN)], win_a)
            pltpu.sync_copy(dsts.at[pl.ds(off, WSCAN)], win_b)
            process(win_a, win_b, base)
            process(win_b, win_a, base)
            return carry

        lax.fori_loop(0, NWIN, window, 0)
        plsc.subcore_barrier()
        pltpu.sync_copy(sp_agg.at[pl.ds(s * share, share)],
                        agg_out.at[pl.ds(base + s * share, share)])
        pltpu.sync_copy(sp_deg.at[pl.ds(s * share, share)], degb)
        pltpu.sync_copy(degb, deg_out.at[pl.ds(base + s * share, share)])
        plsc.subcore_barrier()


def _sc_agg(srcs, dsts, h, x=None):
    from_emb = x is not None
    npc = 3 if from_emb else 2
    rchunk = 8448 if from_emb else 12544
    n2 = NCORES * npc * rchunk
    share = rchunk // NTILES
    ones_arr = jnp.ones((FB,), jnp.float32)
    zagg = jnp.zeros((share, H), jnp.float32)
    zdeg = jnp.zeros((share,), jnp.float32)
    if x is None:
        x = jnp.zeros((8,), jnp.int32)
    scratch = [
        pltpu.VMEM_SHARED((rchunk + DUMP, H), jnp.float32),
        pltpu.VMEM_SHARED((rchunk + DUMP,), jnp.float32),
        pltpu.VMEM((WSCAN,), jnp.int32),
        pltpu.VMEM((WSCAN,), jnp.int32),
        pltpu.VMEM((NBLK, FB), jnp.int32),
        pltpu.VMEM((NBLK, FB), jnp.int32),
        pltpu.VMEM((FB, H), jnp.float32),
        pltpu.VMEM((FB,), jnp.float32),
        pltpu.VMEM((share,), jnp.float32),
        pltpu.SemaphoreType.DMA,
    ]
    scratch += [pltpu.VMEM((FB, H), jnp.float32),
                pltpu.SemaphoreType.DMA,
                pltpu.SemaphoreType.DMA,
                pltpu.SemaphoreType.DMA]
    if from_emb:
        scratch += [pltpu.VMEM_SHARED((T2, H), jnp.float32),
                    pltpu.VMEM((FB,), jnp.int32),
                    pltpu.VMEM((FB,), jnp.int32),
                    pltpu.SemaphoreType.DMA]
    mesh = plsc.VectorSubcoreMesh(core_axis_name="c", subcore_axis_name="s")
    f = pl.kernel(
        functools.partial(_sc_agg_body, from_emb, npc, rchunk),
        out_type=(jax.ShapeDtypeStruct((n2, H), jnp.float32),
                  jax.ShapeDtypeStruct((n2,), jnp.float32)),
        mesh=mesh,
        compiler_params=pltpu.CompilerParams(needs_layout_passes=False),
        scratch_types=scratch,
    )
    return f(srcs, dsts, h, ones_arr, zagg, zdeg, x)


BN = 400
NB = N // BN
T2 = 304


def _embed_body(x_ref, emb_ref, o_ref):
    xb = x_ref[0]
    it = lax.broadcasted_iota(jnp.int32, (T2, BN), 0)
    oh = (it == xb).astype(jnp.float32)
    o_ref[...] = lax.dot_general(oh, emb_ref[...], (((0,), (0,)), ((), ())),
                                 preferred_element_type=jnp.float32)


def _embed(x, emb):
    x3 = x.reshape(NB, 1, BN)
    emb_p = jnp.pad(emb, ((0, T2 - T), (0, 0)))
    return pl.pallas_call(
        _embed_body,
        grid=(NB,),
        in_specs=[pl.BlockSpec((1, 1, BN), lambda i: (i, 0, 0)),
                  pl.BlockSpec((T2, H), lambda i: (0, 0))],
        out_specs=pl.BlockSpec((BN, H), lambda i: (i, 0)),
        out_shape=jax.ShapeDtypeStruct((N, H), jnp.float32),
    )(x3, emb_p)


def _layer_body(agg_ref, deg_ref, h_ref, w_ref, b_ref, o_ref):
    a = agg_ref[...] * lax.rsqrt(jnp.clip(deg_ref[...], 1.0, None))
    z = lax.dot_general(a, w_ref[...], (((1,), (1,)), ((), ())),
                        preferred_element_type=jnp.float32)
    z = z + b_ref[...] + h_ref[...]
    o_ref[...] = 0.5 * z * (1.0 + lax.erf(z * (1.0 / math.sqrt(2.0))))


def _layer(agg_p, deg_p, h, w, b):
    return pl.pallas_call(
        _layer_body,
        grid=(NB,),
        in_specs=[pl.BlockSpec((BN, H), lambda i: (i, 0)),
                  pl.BlockSpec((BN, 1), lambda i: (i, 0)),
                  pl.BlockSpec((BN, H), lambda i: (i, 0)),
                  pl.BlockSpec((H, H), lambda i: (0, 0)),
                  pl.BlockSpec((1, H), lambda i: (0, 0))],
        out_specs=pl.BlockSpec((BN, H), lambda i: (i, 0)),
        out_shape=jax.ShapeDtypeStruct((N, H), jnp.float32),
    )(agg_p, deg_p.reshape(-1, 1), h, w, b.reshape(1, H))


def _pool_body(agg_ref, deg_ref, h_ref, w_ref, b_ref, batch_ref, wo_ref,
               bo_ref, g_ref, be_ref, o_ref, acc, cnt):
    i = pl.program_id(0)

    @pl.when(i == 0)
    def _():
        acc[...] = jnp.zeros_like(acc)
        cnt[...] = jnp.zeros_like(cnt)

    a = agg_ref[...] * lax.rsqrt(jnp.clip(deg_ref[...], 1.0, None))
    z = lax.dot_general(a, w_ref[...], (((1,), (1,)), ((), ())),
                        preferred_element_type=jnp.float32)
    z = z + b_ref[...] + h_ref[...]
    h2 = 0.5 * z * (1.0 + lax.erf(z * (1.0 / math.sqrt(2.0))))

    bb = batch_ref[0]
    seg = lax.broadcasted_iota(jnp.int32, (B, BN), 0)
    m = (seg == bb).astype(jnp.float32)
    acc[...] += lax.dot_general(m, h2, (((1,), (0,)), ((), ())),
                                preferred_element_type=jnp.float32)
    cnt[...] += jnp.sum(m, axis=1, keepdims=True)

    @pl.when(i == NB - 1)
    def _():
        gf = acc[...] / jnp.clip(cnt[...], 1.0, None)
        o = lax.dot_general(gf, wo_ref[...], (((1,), (1,)), ((), ())),
                            preferred_element_type=jnp.float32) + bo_ref[...]
        mu = jnp.mean(o, axis=1, keepdims=True)
        var = jnp.mean((o - mu) ** 2, axis=1, keepdims=True)
        o_ref[...] = (o - mu) * lax.rsqrt(var + 1e-5) * g_ref[...] + be_ref[...]


def _pool(agg_p, deg_p, h1, w, b, batch, wo, bo, gamma, beta):
    batch3 = batch.reshape(NB, 1, BN)
    return pl.pallas_call(
        _pool_body,
        grid=(NB,),
        in_specs=[pl.BlockSpec((BN, H), lambda i: (i, 0)),
                  pl.BlockSpec((BN, 1), lambda i: (i, 0)),
                  pl.BlockSpec((BN, H), lambda i: (i, 0)),
                  pl.BlockSpec((H, H), lambda i: (0, 0)),
                  pl.BlockSpec((1, H), lambda i: (0, 0)),
                  pl.BlockSpec((1, 1, BN), lambda i: (i, 0, 0)),
                  pl.BlockSpec((H, H), lambda i: (0, 0)),
                  pl.BlockSpec((1, H), lambda i: (0, 0)),
                  pl.BlockSpec((1, H), lambda i: (0, 0)),
                  pl.BlockSpec((1, H), lambda i: (0, 0))],
        out_specs=pl.BlockSpec((B, H), lambda i: (0, 0)),
        out_shape=jax.ShapeDtypeStruct((B, H), jnp.float32),
        scratch_shapes=[pltpu.VMEM((B, H), jnp.float32),
                        pltpu.VMEM((B, 1), jnp.float32)],
    )(agg_p, deg_p.reshape(-1, 1), h1, w, b.reshape(1, H), batch3, wo,
      bo.reshape(1, H), gamma.reshape(1, H), beta.reshape(1, H))


def kernel(x, edge_index, batch, batch_size, emb, W1, b1, W2, b2, Wo, bo,
           gamma, beta):
    srcs = edge_index[0]
    dsts = edge_index[1]
    h0 = _embed(x, emb)
    agg1, deg = _sc_agg(srcs, dsts, h0)
    h1 = _layer(agg1, deg, h0, W1, b1)
    agg2, _ = _sc_agg(srcs, dsts, h1)
    return _pool(agg2, deg, h1, W2, b2, batch, Wo, bo, gamma, beta)
